# baseline (speedup 1.0000x reference)
import jax
import jax.numpy as jnp
from jax.experimental import pallas as pl
from jax.experimental.pallas import tpu as pltpu

M = 1024
K = 150528
N_EXP = 64
M_BLK = 16
NUM_M = M // M_BLK


def _router_kernel(x_ref, w_ref, b_ref, out_ref):
    xb = x_ref[...].astype(jnp.bfloat16)
    logits = jax.lax.dot_general(
        xb, w_ref[...],
        (((1,), (1,)), ((), ())),
        preferred_element_type=jnp.float32,
    ) + b_ref[...]
    iota = jax.lax.broadcasted_iota(jnp.int32, logits.shape, 1)
    mx = jnp.max(logits, axis=1, keepdims=True)
    idx = jnp.min(jnp.where(logits == mx, iota, N_EXP),
                  axis=1, keepdims=True)
    out_ref[...] = idx


def kernel(x, W, b):
    x_flat = x.reshape(M, K)
    Wb = W.astype(jnp.bfloat16)
    b2 = b.reshape(1, N_EXP)
    out = pl.pallas_call(
        _router_kernel,
        grid=(NUM_M,),
        in_specs=[
            pl.BlockSpec((M_BLK, K), lambda m: (m, 0)),
            pl.BlockSpec((N_EXP, K), lambda m: (0, 0)),
            pl.BlockSpec((1, N_EXP), lambda m: (0, 0)),
        ],
        out_specs=pl.BlockSpec((M_BLK, 1), lambda m: (m, 0)),
        out_shape=jax.ShapeDtypeStruct((M, 1), jnp.int32),
        compiler_params=pltpu.CompilerParams(
            dimension_semantics=("arbitrary",),
        ),
    )(x_flat, Wb, b2)
    return out.reshape(M)


# speedup vs baseline: 1.1913x; 1.1913x over previous
import jax
import jax.numpy as jnp
from jax.experimental import pallas as pl
from jax.experimental.pallas import tpu as pltpu

M = 1024
K = 150528
N_EXP = 64
NSTREAM = 8
K_CHUNK = K // NSTREAM
M_BLK = 32
NUM_M = M // M_BLK


def _router_kernel(*refs):
    x_refs = refs[:NSTREAM]
    w_ref, b_ref, out_ref = refs[NSTREAM:]
    logits = b_ref[...].astype(jnp.float32)
    for j in range(NSTREAM):
        xb = x_refs[j][...].astype(jnp.bfloat16)
        wj = w_ref[:, j * K_CHUNK:(j + 1) * K_CHUNK]
        logits = logits + jax.lax.dot_general(
            xb, wj,
            (((1,), (1,)), ((), ())),
            preferred_element_type=jnp.float32,
        )
    iota = jax.lax.broadcasted_iota(jnp.int32, logits.shape, 1)
    mx = jnp.max(logits, axis=1, keepdims=True)
    idx = jnp.min(jnp.where(logits == mx, iota, N_EXP),
                  axis=1, keepdims=True)
    out_ref[...] = idx


def kernel(x, W, b):
    x_flat = x.reshape(M, K)
    Wb = W.astype(jnp.bfloat16)
    b2 = b.reshape(1, N_EXP)

    def x_spec(j):
        return pl.BlockSpec((M_BLK, K_CHUNK), lambda m, j=j: (m, j))

    out = pl.pallas_call(
        _router_kernel,
        grid=(NUM_M,),
        in_specs=[x_spec(j) for j in range(NSTREAM)] + [
            pl.BlockSpec((N_EXP, K), lambda m: (0, 0)),
            pl.BlockSpec((1, N_EXP), lambda m: (0, 0)),
        ],
        out_specs=pl.BlockSpec((M_BLK, 1), lambda m: (m, 0)),
        out_shape=jax.ShapeDtypeStruct((M, 1), jnp.int32),
        compiler_params=pltpu.CompilerParams(
            dimension_semantics=("arbitrary",),
            vmem_limit_bytes=100 * 1024 * 1024,
        ),
    )(*([x_flat] * NSTREAM), Wb, b2)
    return out.reshape(M)


# speedup vs baseline: 1.1931x; 1.0015x over previous
import jax
import jax.numpy as jnp
from jax.experimental import pallas as pl
from jax.experimental.pallas import tpu as pltpu

M = 1024
K = 150528
N_EXP = 64
NSTREAM = 8
K_CHUNK = K // NSTREAM
M_BLK = 32
NUM_M = M // M_BLK


def _router_kernel(*refs):
    x_refs = refs[:NSTREAM]
    w_ref, b_ref, out_ref = refs[NSTREAM:]
    logits = b_ref[...].astype(jnp.float32)
    for j in range(NSTREAM):
        xb = x_refs[j][...].astype(jnp.bfloat16)
        wj = w_ref[:, j * K_CHUNK:(j + 1) * K_CHUNK]
        logits = logits + jax.lax.dot_general(
            xb, wj,
            (((1,), (1,)), ((), ())),
            preferred_element_type=jnp.float32,
        )
    iota = jax.lax.broadcasted_iota(jnp.int32, logits.shape, 1)
    mx = jnp.max(logits, axis=1, keepdims=True)
    idx = jnp.min(jnp.where(logits == mx, iota, N_EXP),
                  axis=1, keepdims=True)
    out_ref[...] = idx


def kernel(x, W, b):
    x_flat = x.reshape(M, K)
    Wb = W.astype(jnp.bfloat16)
    b2 = b.reshape(1, N_EXP)

    def x_spec(j):
        return pl.BlockSpec((M_BLK, K_CHUNK), lambda m, j=j: (m, j))

    out = pl.pallas_call(
        _router_kernel,
        grid=(NUM_M,),
        in_specs=[x_spec(j) for j in range(NSTREAM)] + [
            pl.BlockSpec((N_EXP, K), lambda m: (0, 0)),
            pl.BlockSpec((1, N_EXP), lambda m: (0, 0)),
        ],
        out_specs=pl.BlockSpec((M_BLK, 1), lambda m: (m, 0)),
        out_shape=jax.ShapeDtypeStruct((M, 1), jnp.int32),
        compiler_params=pltpu.CompilerParams(
            dimension_semantics=("parallel",),
            vmem_limit_bytes=100 * 1024 * 1024,
        ),
    )(*([x_flat] * NSTREAM), Wb, b2)
    return out.reshape(M)


# speedup vs baseline: 4.6206x; 3.8726x over previous
import jax
import jax.numpy as jnp
from jax.experimental import pallas as pl
from jax.experimental.pallas import tpu as pltpu

M = 1024
K = 150528
N_EXP = 64
K_BLK = 3072
NUM_K = K // K_BLK


def _router_kernel(xt_ref, w_ref, b_ref, out_ref, acc_ref):
    k = pl.program_id(0)
    part = jax.lax.dot_general(
        w_ref[...].astype(jnp.bfloat16),
        xt_ref[...].astype(jnp.bfloat16),
        (((1,), (0,)), ((), ())),
        preferred_element_type=jnp.float32,
    )

    @pl.when(k == 0)
    def _init():
        acc_ref[...] = part + b_ref[...]

    @pl.when(k > 0)
    def _accum():
        acc_ref[...] += part

    @pl.when(k == NUM_K - 1)
    def _finish():
        acc = acc_ref[...]
        iota = jax.lax.broadcasted_iota(jnp.int32, acc.shape, 0)
        mx = jnp.max(acc, axis=0, keepdims=True)
        idx = jnp.min(jnp.where(acc == mx, iota, N_EXP),
                      axis=0, keepdims=True)
        out_ref[...] = idx


def kernel(x, W, b):
    xt = x.transpose(1, 2, 3, 0).reshape(K, M)
    b2 = b.reshape(N_EXP, 1)
    out = pl.pallas_call(
        _router_kernel,
        grid=(NUM_K,),
        in_specs=[
            pl.BlockSpec((K_BLK, M), lambda k: (k, 0)),
            pl.BlockSpec((N_EXP, K_BLK), lambda k: (0, k)),
            pl.BlockSpec((N_EXP, 1), lambda k: (0, 0)),
        ],
        out_specs=pl.BlockSpec((1, M), lambda k: (0, 0)),
        out_shape=jax.ShapeDtypeStruct((1, M), jnp.int32),
        scratch_shapes=[pltpu.VMEM((N_EXP, M), jnp.float32)],
        compiler_params=pltpu.CompilerParams(
            dimension_semantics=("arbitrary",),
        ),
    )(xt, W, b2)
    return out.reshape(M)
